# baseline (speedup 1.0000x reference)
import functools

import jax
import jax.numpy as jnp
from jax import lax
from jax.experimental import pallas as pl
from jax.experimental.pallas import tpu as pltpu
from jax.experimental.pallas import tpu_sc as plsc

D = 1024
SEQ_BLOCK = 512


def _gather_body(ids_hbm, g_hbm, b_hbm, go_hbm, bo_hbm, ids_v, gv, bv, sem):
    wid = lax.axis_index("s") * 2 + lax.axis_index("c")

    @pl.when(wid == 0)
    def _():
        pltpu.sync_copy(ids_hbm, ids_v)
        pltpu.async_copy(g_hbm.at[ids_v], gv, sem).wait()
        pltpu.async_copy(b_hbm.at[ids_v], bv, sem).wait()
        pltpu.sync_copy(gv, go_hbm)
        pltpu.sync_copy(bv, bo_hbm)


def _sc_gather(ids, gamma, beta):
    n, d = gamma.shape
    mesh = plsc.VectorSubcoreMesh(core_axis_name="c", subcore_axis_name="s")
    return pl.kernel(
        _gather_body,
        out_type=(
            jax.ShapeDtypeStruct((n, d), gamma.dtype),
            jax.ShapeDtypeStruct((n, d), beta.dtype),
        ),
        mesh=mesh,
        scratch_types=[
            pltpu.VMEM((n,), jnp.int32),
            pltpu.VMEM((n, d), jnp.float32),
            pltpu.VMEM((n, d), jnp.float32),
            pltpu.SemaphoreType.DMA,
        ],
    )(ids, gamma, beta)


def _film_body(x_ref, g_ref, b_ref, o_ref):
    o_ref[...] = g_ref[...] * x_ref[...] + b_ref[...]


@jax.jit
def _film(x, ids, gamma, beta):
    B, S, Dm = x.shape
    g_rows, b_rows = _sc_gather(ids, gamma, beta)
    g3 = g_rows.reshape(B, 1, Dm)
    b3 = b_rows.reshape(B, 1, Dm)
    return pl.pallas_call(
        _film_body,
        grid=(B, S // SEQ_BLOCK),
        in_specs=[
            pl.BlockSpec((1, SEQ_BLOCK, Dm), lambda b, s: (b, s, 0)),
            pl.BlockSpec((1, 1, Dm), lambda b, s: (b, 0, 0)),
            pl.BlockSpec((1, 1, Dm), lambda b, s: (b, 0, 0)),
        ],
        out_specs=pl.BlockSpec((1, SEQ_BLOCK, Dm), lambda b, s: (b, s, 0)),
        out_shape=jax.ShapeDtypeStruct((B, S, Dm), x.dtype),
        compiler_params=pltpu.CompilerParams(
            dimension_semantics=("parallel", "arbitrary"),
        ),
    )(x, g3, b3)


def kernel(x, condition_ids, gamma, beta):
    return _film(x, condition_ids.astype(jnp.int32), gamma, beta)


# speedup vs baseline: 1.0900x; 1.0900x over previous
import functools

import jax
import jax.numpy as jnp
from jax import lax
from jax.experimental import pallas as pl
from jax.experimental.pallas import tpu as pltpu
from jax.experimental.pallas import tpu_sc as plsc

D = 1024
SEQ_BLOCK = 2048


def _gather_body(ids_hbm, g_hbm, b_hbm, go_hbm, bo_hbm, ids_v, gv, bv, sem):
    wid = lax.axis_index("s") * 2 + lax.axis_index("c")

    @pl.when(wid == 0)
    def _():
        pltpu.sync_copy(ids_hbm, ids_v)
        pltpu.async_copy(g_hbm.at[ids_v], gv, sem).wait()
        pltpu.async_copy(b_hbm.at[ids_v], bv, sem).wait()
        pltpu.sync_copy(gv, go_hbm)
        pltpu.sync_copy(bv, bo_hbm)


def _sc_gather(ids, gamma, beta):
    n, d = gamma.shape
    mesh = plsc.VectorSubcoreMesh(core_axis_name="c", subcore_axis_name="s")
    return pl.kernel(
        _gather_body,
        out_type=(
            jax.ShapeDtypeStruct((n, d), gamma.dtype),
            jax.ShapeDtypeStruct((n, d), beta.dtype),
        ),
        mesh=mesh,
        scratch_types=[
            pltpu.VMEM((n,), jnp.int32),
            pltpu.VMEM((n, d), jnp.float32),
            pltpu.VMEM((n, d), jnp.float32),
            pltpu.SemaphoreType.DMA,
        ],
    )(ids, gamma, beta)


def _film_body(x_ref, g_ref, b_ref, o_ref):
    o_ref[...] = g_ref[...] * x_ref[...] + b_ref[...]


@jax.jit
def _film(x, ids, gamma, beta):
    B, S, Dm = x.shape
    g_rows, b_rows = _sc_gather(ids, gamma, beta)
    g3 = g_rows.reshape(B, 1, Dm)
    b3 = b_rows.reshape(B, 1, Dm)
    return pl.pallas_call(
        _film_body,
        grid=(B, S // SEQ_BLOCK),
        in_specs=[
            pl.BlockSpec((1, SEQ_BLOCK, Dm), lambda b, s: (b, s, 0)),
            pl.BlockSpec((1, 1, Dm), lambda b, s: (b, 0, 0)),
            pl.BlockSpec((1, 1, Dm), lambda b, s: (b, 0, 0)),
        ],
        out_specs=pl.BlockSpec((1, SEQ_BLOCK, Dm), lambda b, s: (b, s, 0)),
        out_shape=jax.ShapeDtypeStruct((B, S, Dm), x.dtype),
        compiler_params=pltpu.CompilerParams(
            dimension_semantics=("parallel", "arbitrary"),
        ),
    )(x, g3, b3)


def kernel(x, condition_ids, gamma, beta):
    return _film(x, condition_ids.astype(jnp.int32), gamma, beta)


# speedup vs baseline: 1.1217x; 1.0290x over previous
import jax
import jax.numpy as jnp
from jax import lax
from jax.experimental import pallas as pl
from jax.experimental.pallas import tpu as pltpu
from jax.experimental.pallas import tpu_sc as plsc

D = 1024
ROWS = 1024
NBUF = 4


def _gather_body(ids_hbm, g_hbm, b_hbm, go_hbm, bo_hbm, ids_v, gv, bv, sem):
    wid = lax.axis_index("s") * 2 + lax.axis_index("c")

    @pl.when(wid == 0)
    def _():
        pltpu.sync_copy(ids_hbm, ids_v)
        pltpu.async_copy(g_hbm.at[ids_v], gv, sem).wait()
        pltpu.async_copy(b_hbm.at[ids_v], bv, sem).wait()
        pltpu.sync_copy(gv, go_hbm)
        pltpu.sync_copy(bv, bo_hbm)


def _sc_gather(ids, gamma, beta):
    n, d = gamma.shape
    mesh = plsc.VectorSubcoreMesh(core_axis_name="c", subcore_axis_name="s")
    return pl.kernel(
        _gather_body,
        out_type=(
            jax.ShapeDtypeStruct((n, d), gamma.dtype),
            jax.ShapeDtypeStruct((n, d), beta.dtype),
        ),
        mesh=mesh,
        scratch_types=[
            pltpu.VMEM((n,), jnp.int32),
            pltpu.VMEM((n, d), jnp.float32),
            pltpu.VMEM((n, d), jnp.float32),
            pltpu.SemaphoreType.DMA,
        ],
    )(ids, gamma, beta)


def _film_body(x_hbm, g_ref, b_ref, o_hbm, buf, in_sems, out_sems):
    n_rows = x_hbm.shape[0]
    n_chunks = n_rows // ROWS
    rows_per_batch = n_rows // g_ref.shape[0]

    def start_in(c, bi):
        cp = pltpu.make_async_copy(
            x_hbm.at[pl.ds(c * ROWS, ROWS), :], buf.at[bi], in_sems.at[bi])
        cp.start()
        return cp

    def start_out(c, bi):
        cp = pltpu.make_async_copy(
            buf.at[bi], o_hbm.at[pl.ds(c * ROWS, ROWS), :], out_sems.at[bi])
        cp.start()
        return cp

    in_cp = [None] * NBUF
    out_cp = [None] * NBUF
    for c in range(min(NBUF - 1, n_chunks)):
        in_cp[c] = start_in(c, c)
    for c in range(n_chunks):
        bi = c % NBUF
        batch = (c * ROWS) // rows_per_batch
        in_cp[bi].wait()
        buf[bi] = g_ref[batch] * buf[bi] + b_ref[batch]
        out_cp[bi] = start_out(c, bi)
        nxt = c + NBUF - 1
        if nxt < n_chunks:
            nbi = nxt % NBUF
            if out_cp[nbi] is not None:
                out_cp[nbi].wait()
            in_cp[nbi] = start_in(nxt, nbi)
    for cp in out_cp:
        if cp is not None:
            cp.wait()


@jax.jit
def _film(x, ids, gamma, beta):
    B, S, Dm = x.shape
    g_rows, b_rows = _sc_gather(ids, gamma, beta)
    x2d = x.reshape(B * S, Dm)
    out2d = pl.pallas_call(
        _film_body,
        in_specs=[
            pl.BlockSpec(memory_space=pl.ANY),
            pl.BlockSpec(memory_space=pltpu.MemorySpace.VMEM),
            pl.BlockSpec(memory_space=pltpu.MemorySpace.VMEM),
        ],
        out_specs=pl.BlockSpec(memory_space=pl.ANY),
        out_shape=jax.ShapeDtypeStruct((B * S, Dm), x.dtype),
        scratch_shapes=[
            pltpu.VMEM((NBUF, ROWS, Dm), jnp.float32),
            pltpu.SemaphoreType.DMA((NBUF,)),
            pltpu.SemaphoreType.DMA((NBUF,)),
        ],
    )(x2d, g_rows, b_rows)
    return out2d.reshape(B, S, Dm)


def kernel(x, condition_ids, gamma, beta):
    return _film(x, condition_ids.astype(jnp.int32), gamma, beta)
